# baseline (speedup 1.0000x reference)
import jax
import jax.numpy as jnp
from jax.experimental import pallas as pl
from jax.experimental.pallas import tpu as pltpu

_EPS = 1e-5
_K = 9
_CPAD1 = 24
_CPAD2 = 56


def _fold_and_pad(w1, b1, g1, be1, m1, v1, w2, b2, g2, be2, m2, v2):
    G, K, Cs, C1 = w1.shape
    C2 = w2.shape[-1]
    Cin = G * Cs
    C1m = G * C1

    s1 = g1 * jax.lax.rsqrt(v1 + _EPS)
    w1f = w1 * s1[:, None, None, :]
    w1full = jnp.zeros((K, Cin, C1m), jnp.float32)
    for g in range(G):
        w1full = w1full.at[:, g * Cs:(g + 1) * Cs, g * C1:(g + 1) * C1].set(
            w1f[g])
    w1full = jnp.pad(w1full, ((0, 0), (0, _CPAD1 - Cin), (0, 0)))
    w1p = w1full.reshape(K * _CPAD1, C1m).T
    b1p = ((b1 - m1) * s1 + be1).reshape(C1m, 1)

    s2 = g2 * jax.lax.rsqrt(v2 + _EPS)
    w2full = jnp.transpose(w2, (1, 0, 2, 3)).reshape(K, C1m, C2) * s2[None]
    w2full = jnp.pad(w2full, ((0, 0), (0, _CPAD2 - C1m), (0, 0)))
    w2p = w2full.reshape(K * _CPAD2, C2).T
    b2p = ((b2 - m2) * s2 + be2).reshape(C2, 1)
    return w1p, b1p, w2p, b2p


def _rot(a, k):
    if k == 0:
        return a
    return jnp.concatenate([a[:, k:], a[:, :k]], axis=1)


def _conv_body(x_ref, w1_ref, b1_ref, w2_ref, b2_ref, o_ref):
    B, C2, L2 = o_ref.shape
    Cin = x_ref.shape[0]
    L = x_ref.shape[2]
    BL = B * L

    x3 = jnp.transpose(x_ref[...], (1, 0, 2))
    x = jnp.concatenate([x3[b] for b in range(B)], axis=1)
    xp = jnp.concatenate(
        [x, jnp.zeros((_CPAD1 - Cin, BL), x.dtype)], axis=0)

    cols1 = jnp.concatenate(
        [_rot(xp, k) for k in range(_K)], axis=0).astype(jnp.bfloat16)
    h = jnp.dot(w1_ref[...], cols1, preferred_element_type=jnp.float32)
    h = jnp.maximum(h + b1_ref[...], 0.0)

    hp = jnp.concatenate(
        [h, jnp.zeros((_CPAD2 - h.shape[0], BL), h.dtype)], axis=0)

    cols2 = jnp.concatenate(
        [_rot(hp, k) for k in range(_K)], axis=0).astype(jnp.bfloat16)
    z = jnp.dot(w2_ref[...], cols2, preferred_element_type=jnp.float32)
    z = jnp.maximum(z + b2_ref[...], 0.0)

    for b in range(B):
        o_ref[b] = z[:, b * L:b * L + L2]


def kernel(x, w1, b1, g1, be1, m1, v1, w2, b2, g2, be2, m2, v2):
    w1p, b1p, w2p, b2p = _fold_and_pad(
        w1, b1, g1, be1, m1, v1, w2, b2, g2, be2, m2, v2)
    w1p = w1p.astype(jnp.bfloat16)
    w2p = w2p.astype(jnp.bfloat16)

    N, Cin, L = x.shape
    C2 = w2p.shape[0]
    L2 = L - 2 * (_K - 1)

    xt = jnp.transpose(x, (1, 0, 2))

    B = 32
    while N % B:
        B //= 2
    grid = (N // B,)

    return pl.pallas_call(
        _conv_body,
        out_shape=jax.ShapeDtypeStruct((N, C2, L2), x.dtype),
        grid_spec=pltpu.PrefetchScalarGridSpec(
            num_scalar_prefetch=0,
            grid=grid,
            in_specs=[
                pl.BlockSpec((Cin, B, L), lambda n: (0, n, 0)),
                pl.BlockSpec(w1p.shape, lambda n: (0, 0)),
                pl.BlockSpec(b1p.shape, lambda n: (0, 0)),
                pl.BlockSpec(w2p.shape, lambda n: (0, 0)),
                pl.BlockSpec(b2p.shape, lambda n: (0, 0)),
            ],
            out_specs=pl.BlockSpec((B, C2, L2), lambda n: (n, 0, 0)),
        ),
        compiler_params=pltpu.CompilerParams(
            dimension_semantics=("parallel",),
            vmem_limit_bytes=56 << 20),
    )(xt, w1p, b1p, w2p, b2p)


# speedup vs baseline: 1.0017x; 1.0017x over previous
import jax
import jax.numpy as jnp
from jax.experimental import pallas as pl
from jax.experimental.pallas import tpu as pltpu

_EPS = 1e-5
_K = 9
_CPAD1 = 24
_CPAD2 = 56


def _fold_and_pad(w1, b1, g1, be1, m1, v1, w2, b2, g2, be2, m2, v2):
    G, K, Cs, C1 = w1.shape
    C2 = w2.shape[-1]
    Cin = G * Cs
    C1m = G * C1

    s1 = g1 * jax.lax.rsqrt(v1 + _EPS)
    w1f = w1 * s1[:, None, None, :]
    w1full = jnp.zeros((K, Cin, C1m), jnp.float32)
    for g in range(G):
        w1full = w1full.at[:, g * Cs:(g + 1) * Cs, g * C1:(g + 1) * C1].set(
            w1f[g])
    w1full = jnp.pad(w1full, ((0, 0), (0, _CPAD1 - Cin), (0, 0)))
    w1p = w1full.reshape(K * _CPAD1, C1m).T
    b1p = ((b1 - m1) * s1 + be1).reshape(C1m, 1)

    s2 = g2 * jax.lax.rsqrt(v2 + _EPS)
    w2full = jnp.transpose(w2, (1, 0, 2, 3)).reshape(K, C1m, C2) * s2[None]
    w2full = jnp.pad(w2full, ((0, 0), (0, _CPAD2 - C1m), (0, 0)))
    w2p = w2full.reshape(K * _CPAD2, C2).T
    b2p = ((b2 - m2) * s2 + be2).reshape(C2, 1)
    return w1p, b1p, w2p, b2p


def _rot(a, k):
    if k == 0:
        return a
    return jnp.concatenate([a[:, k:], a[:, :k]], axis=1)


def _conv_body(x_ref, w1_ref, b1_ref, w2_ref, b2_ref, o_ref, c1_ref, c2_ref):
    B, C2, L2 = o_ref.shape
    Cin = x_ref.shape[0]
    L = x_ref.shape[2]
    BL = B * L

    x3 = jnp.transpose(x_ref[...], (1, 0, 2))
    x = jnp.concatenate([x3[b] for b in range(B)], axis=1)
    xp = jnp.concatenate(
        [x, jnp.zeros((_CPAD1 - Cin, BL), x.dtype)], axis=0)

    c1_ref[...] = jnp.concatenate([_rot(xp, k) for k in range(_K)], axis=0)
    h = jnp.dot(w1_ref[...], c1_ref[...].astype(jnp.bfloat16),
                preferred_element_type=jnp.float32)
    h = jnp.maximum(h + b1_ref[...], 0.0)

    hp = jnp.concatenate(
        [h, jnp.zeros((_CPAD2 - h.shape[0], BL), h.dtype)], axis=0)

    c2_ref[...] = jnp.concatenate([_rot(hp, k) for k in range(_K)], axis=0)
    z = jnp.dot(w2_ref[...], c2_ref[...].astype(jnp.bfloat16),
                preferred_element_type=jnp.float32)
    z = jnp.maximum(z + b2_ref[...], 0.0)

    for b in range(B):
        o_ref[b] = z[:, b * L:b * L + L2]


def kernel(x, w1, b1, g1, be1, m1, v1, w2, b2, g2, be2, m2, v2):
    w1p, b1p, w2p, b2p = _fold_and_pad(
        w1, b1, g1, be1, m1, v1, w2, b2, g2, be2, m2, v2)
    w1p = w1p.astype(jnp.bfloat16)
    w2p = w2p.astype(jnp.bfloat16)

    N, Cin, L = x.shape
    C2 = w2p.shape[0]
    L2 = L - 2 * (_K - 1)

    xt = jnp.transpose(x, (1, 0, 2))

    B = 16
    while N % B:
        B //= 2
    grid = (N // B,)

    return pl.pallas_call(
        _conv_body,
        out_shape=jax.ShapeDtypeStruct((N, C2, L2), x.dtype),
        grid_spec=pltpu.PrefetchScalarGridSpec(
            num_scalar_prefetch=0,
            grid=grid,
            in_specs=[
                pl.BlockSpec((Cin, B, L), lambda n: (0, n, 0)),
                pl.BlockSpec(w1p.shape, lambda n: (0, 0)),
                pl.BlockSpec(b1p.shape, lambda n: (0, 0)),
                pl.BlockSpec(w2p.shape, lambda n: (0, 0)),
                pl.BlockSpec(b2p.shape, lambda n: (0, 0)),
            ],
            out_specs=pl.BlockSpec((B, C2, L2), lambda n: (n, 0, 0)),
            scratch_shapes=[
                pltpu.VMEM((_K * _CPAD1, B * L), jnp.float32),
                pltpu.VMEM((_K * _CPAD2, B * L), jnp.float32),
            ],
        ),
        compiler_params=pltpu.CompilerParams(
            dimension_semantics=("parallel",),
            vmem_limit_bytes=56 << 20),
    )(xt, w1p, b1p, w2p, b2p)
